# baseline (speedup 1.0000x reference)
import jax
import jax.numpy as jnp
from jax import lax
from jax.experimental import pallas as pl
from jax.experimental.pallas import tpu as pltpu
from jax.experimental.pallas import tpu_sc as plsc

N = 320000
D = 128
DH = D // 2
S = 10000
NC, NS = 2, 16
L = 16
BLK = 128
NBLK = N // BLK
BPS = (NBLK + NS - 1) // NS
NBLK_PAD = NS * BPS
TROWS = S // NS
EPS = 1e-9


LASTB = NBLK - (NS - 1) * BPS


def _body(x_hbm, idx_hbm, zeros_hbm, ones_hbm, out_hbm,
          buf, cnt, idxb, ones_v, table_sh, cnt_sh,
          sa0, sa1, sw0, sw1, so):
    c = lax.axis_index("c")
    s = lax.axis_index("s")
    t0 = s * TROWS
    blk0 = s * BPS
    nblk = jnp.minimum(BPS, NBLK - blk0)
    sa = (sa0, sa1)
    sw = (sw0, sw1)

    @pl.when(s < NS - 1)
    def _():
        pltpu.sync_copy(idx_hbm.at[pl.ds(blk0, BPS), c], idxb)

    @pl.when(s == NS - 1)
    def _():
        pltpu.sync_copy(idx_hbm.at[pl.ds(blk0, LASTB), c],
                        idxb.at[pl.ds(0, LASTB)])

    pltpu.sync_copy(ones_hbm, ones_v)

    def x_slice(k, h):
        return x_hbm.at[pl.ds((blk0 + k) * BLK, BLK), pl.ds(h * DH, DH)]

    def slot(p):
        return buf.at[pl.ds(p * BLK, BLK)]

    def out_slice(k, h):
        return out_hbm.at[pl.ds((blk0 + k) * BLK, BLK), c, pl.ds(h * DH, DH)]

    for h in range(2):
        pltpu.sync_copy(zeros_hbm.at[pl.ds(t0, TROWS)],
                        table_sh.at[pl.ds(t0, TROWS)])
        if h == 0:
            pltpu.sync_copy(zeros_hbm.at[pl.ds(t0, TROWS), pl.ds(0, L)],
                            cnt_sh.at[pl.ds(t0, TROWS)])
        plsc.subcore_barrier()

        pltpu.async_copy(x_slice(0, h), slot(0), sa[0])

        def p1(k2, carry):
            for par in range(2):
                k = 2 * k2 + par

                @pl.when(k < nblk)
                def _():
                    pltpu.make_async_copy(x_slice(k, h), slot(par),
                                          sa[par]).wait()

                    @pl.when(k + 1 < nblk)
                    def _():
                        pltpu.async_copy(x_slice(k + 1, h), slot(1 - par),
                                         sa[1 - par])

                    if h == 0:
                        pltpu.async_copy(ones_v, cnt_sh.at[idxb.at[k]], so,
                                         add=True)
                    pltpu.sync_copy(slot(par), table_sh.at[idxb.at[k]],
                                    add=True)
            return carry

        lax.fori_loop(0, (nblk + 1) // 2, p1, 0)
        if h == 0:
            def drain(k, carry):
                pltpu.make_async_copy(ones_v, cnt_sh.at[idxb.at[k]], so).wait()
                return carry

            lax.fori_loop(0, nblk, drain, 0)
        plsc.subcore_barrier()

        pltpu.sync_copy(table_sh.at[pl.ds(t0, TROWS)], buf.at[pl.ds(0, TROWS)])
        if h == 0:
            pltpu.sync_copy(cnt_sh.at[pl.ds(t0, TROWS)], cnt)

        def p2(r, carry):
            rcp = 1.0 / (cnt[r, :] + EPS)
            for g in range(DH // L):
                buf[r, pl.ds(g * L, L)] = buf[r, pl.ds(g * L, L)] * rcp
            return carry

        lax.fori_loop(0, TROWS, p2, 0)
        pltpu.sync_copy(buf.at[pl.ds(0, TROWS)], table_sh.at[pl.ds(t0, TROWS)])
        plsc.subcore_barrier()

        pltpu.async_copy(table_sh.at[idxb.at[0]], slot(0), sa[0])

        def p3(k2, carry):
            for par in range(2):
                k = 2 * k2 + par

                @pl.when(k < nblk)
                def _():
                    pltpu.make_async_copy(table_sh.at[idxb.at[k]], slot(par),
                                          sa[par]).wait()

                    @pl.when(k + 1 < nblk)
                    def _():
                        pltpu.async_copy(table_sh.at[idxb.at[k + 1]],
                                         slot(1 - par), sa[1 - par])

                    @pl.when(k >= 2)
                    def _():
                        pltpu.make_async_copy(slot(par), out_slice(k - 2, h),
                                              sw[par]).wait()

                    pltpu.async_copy(slot(par), out_slice(k, h), sw[par])
            return carry

        lax.fori_loop(0, (nblk + 1) // 2, p3, 0)
        for par in range(2):
            pltpu.make_async_copy(slot(par), out_slice(par, h), sw[par]).wait()
        if h == 0:
            plsc.subcore_barrier()


_mesh = plsc.VectorSubcoreMesh(core_axis_name="c", subcore_axis_name="s",
                               num_cores=NC, num_subcores=NS)

_sc_call = pl.kernel(
    _body,
    out_type=jax.ShapeDtypeStruct((N, NC, D), jnp.float32),
    mesh=_mesh,
    compiler_params=pltpu.CompilerParams(use_tc_tiling_on_sc=False),
    scratch_types=[
        pltpu.VMEM((TROWS, DH), jnp.float32),
        pltpu.VMEM((TROWS, L), jnp.float32),
        pltpu.VMEM((BPS, BLK), jnp.int32),
        pltpu.VMEM((BLK, L), jnp.float32),
        pltpu.VMEM_SHARED((S, DH), jnp.float32),
        pltpu.VMEM_SHARED((S, L), jnp.float32),
        pltpu.SemaphoreType.DMA,
        pltpu.SemaphoreType.DMA,
        pltpu.SemaphoreType.DMA,
        pltpu.SemaphoreType.DMA,
        pltpu.SemaphoreType.DMA,
    ],
)


def kernel(input, index):
    idx2 = jnp.transpose(index.astype(jnp.int32).reshape(NBLK, BLK, NC),
                         (0, 2, 1))
    zeros = jnp.zeros((S, DH), jnp.float32)
    ones = jnp.ones((BLK, L), jnp.float32)
    out = _sc_call(input, idx2, zeros, ones)
    return out.reshape(N, NC * D)


# speedup vs baseline: 1.5117x; 1.5117x over previous
import jax
import jax.numpy as jnp
from jax import lax
from jax.experimental import pallas as pl
from jax.experimental.pallas import tpu as pltpu
from jax.experimental.pallas import tpu_sc as plsc

N = 320000
D = 128
DH = D // 2
S = 10000
NC, NS = 2, 16
L = 16
BLK = 128
NBLK = N // BLK
BPS = (NBLK + NS - 1) // NS
NBLK_PAD = NS * BPS
TROWS = S // NS
P2C = 125
EPS = 1e-9


LASTB = NBLK - (NS - 1) * BPS


def _body(x_hbm, idx_hbm, idxp_hbm, zeros_hbm, ones_hbm, out_hbm,
          buf, cnt, idxb, idxbp, ones_v, table_sh, cnt_sh,
          sa0, sa1, sw0, sw1, so):
    c = lax.axis_index("c")
    s = lax.axis_index("s")
    t0 = s * TROWS
    blk0 = s * BPS
    nblk = jnp.minimum(BPS, NBLK - blk0)
    sa = (sa0, sa1)
    sw = (sw0, sw1)

    @pl.when(s < NS - 1)
    def _():
        pltpu.sync_copy(idx_hbm.at[pl.ds(blk0, BPS), c], idxb)
        pltpu.sync_copy(idxp_hbm.at[pl.ds(blk0, BPS), c], idxbp)

    @pl.when(s == NS - 1)
    def _():
        pltpu.sync_copy(idx_hbm.at[pl.ds(blk0, LASTB), c],
                        idxb.at[pl.ds(0, LASTB)])
        pltpu.sync_copy(idxp_hbm.at[pl.ds(blk0, LASTB), c],
                        idxbp.at[pl.ds(0, LASTB)])

    pltpu.sync_copy(ones_hbm, ones_v)

    def x_slice(k, h):
        return x_hbm.at[pl.ds((blk0 + k) * BLK, BLK), pl.ds(h * DH, DH)]

    def slot(p):
        return buf.at[pl.ds(p * BLK, BLK)]

    def out_slice(k, h, r):
        return out_hbm.at[pl.ds((blk0 + k) * (BLK // 8), BLK // 8), c, r,
                          pl.ds(h * DH, DH)]

    def gslot(p, r):
        return buf.at[pl.ds(p * BLK + r * (BLK // 8), BLK // 8)]

    for h in range(2):
        pltpu.sync_copy(zeros_hbm.at[pl.ds(t0, TROWS)],
                        table_sh.at[pl.ds(t0, TROWS)])
        if h == 0:
            pltpu.sync_copy(zeros_hbm.at[pl.ds(t0, TROWS), pl.ds(0, L)],
                            cnt_sh.at[pl.ds(t0, TROWS)])
        plsc.subcore_barrier()

        pltpu.async_copy(x_slice(0, h), slot(0), sa[0])

        def p1(k2, carry):
            for par in range(2):
                k = 2 * k2 + par

                @pl.when(k < nblk)
                def _():
                    pltpu.make_async_copy(x_slice(k, h), slot(par),
                                          sa[par]).wait()

                    @pl.when(k + 1 < nblk)
                    def _():
                        pltpu.async_copy(x_slice(k + 1, h), slot(1 - par),
                                         sa[1 - par])

                    if h == 0:
                        pltpu.async_copy(ones_v, cnt_sh.at[idxb.at[k]], so,
                                         add=True)
                    pltpu.sync_copy(slot(par), table_sh.at[idxb.at[k]],
                                    add=True)
            return carry

        lax.fori_loop(0, (nblk + 1) // 2, p1, 0)
        if h == 0:
            def drain(k, carry):
                pltpu.make_async_copy(ones_v, cnt_sh.at[idxb.at[k]], so).wait()
                return carry

            lax.fori_loop(0, nblk, drain, 0)
        plsc.subcore_barrier()

        if h == 0:
            pltpu.sync_copy(cnt_sh.at[pl.ds(t0, TROWS)], cnt)
        for q in range(TROWS // P2C):
            pltpu.sync_copy(table_sh.at[pl.ds(t0 + q * P2C, P2C)],
                            buf.at[pl.ds(0, P2C)])

            def p2(r, carry):
                rcp = 1.0 / (cnt[q * P2C + r, :] + EPS)
                for g in range(DH // L):
                    buf[r, pl.ds(g * L, L)] = buf[r, pl.ds(g * L, L)] * rcp
                return carry

            lax.fori_loop(0, P2C, p2, 0)
            pltpu.sync_copy(buf.at[pl.ds(0, P2C)],
                            table_sh.at[pl.ds(t0 + q * P2C, P2C)])
        plsc.subcore_barrier()

        pltpu.async_copy(table_sh.at[idxbp.at[0]], slot(0), sa[0])

        def p3(k2, carry):
            for par in range(2):
                k = 2 * k2 + par

                @pl.when(k < nblk)
                def _():
                    pltpu.make_async_copy(table_sh.at[idxbp.at[k]], slot(par),
                                          sa[par]).wait()

                    @pl.when(k + 1 < nblk)
                    def _():
                        pltpu.async_copy(table_sh.at[idxbp.at[k + 1]],
                                         slot(1 - par), sa[1 - par])

                    @pl.when(k >= 2)
                    def _():
                        for r in range(8):
                            pltpu.make_async_copy(
                                gslot(par, r), out_slice(k - 2, h, r),
                                sw[par]).wait()

                    for r in range(8):
                        pltpu.async_copy(gslot(par, r), out_slice(k, h, r),
                                         sw[par])
            return carry

        lax.fori_loop(0, (nblk + 1) // 2, p3, 0)
        for par in range(2):
            for r in range(8):
                pltpu.make_async_copy(gslot(par, r), out_slice(par, h, r),
                                      sw[par]).wait()
        if h == 0:
            plsc.subcore_barrier()


_mesh = plsc.VectorSubcoreMesh(core_axis_name="c", subcore_axis_name="s",
                               num_cores=NC, num_subcores=NS)

_sc_call = pl.kernel(
    _body,
    out_type=jax.ShapeDtypeStruct((N // 8, NC, 8, D), jnp.float32),
    mesh=_mesh,
    compiler_params=pltpu.CompilerParams(use_tc_tiling_on_sc=False),
    scratch_types=[
        pltpu.VMEM((2 * BLK, DH), jnp.float32),
        pltpu.VMEM((TROWS, L), jnp.float32),
        pltpu.VMEM((BPS, BLK), jnp.int32),
        pltpu.VMEM((BPS, BLK), jnp.int32),
        pltpu.VMEM((BLK, L), jnp.float32),
        pltpu.VMEM_SHARED((S, DH), jnp.float32),
        pltpu.VMEM_SHARED((S, L), jnp.float32),
        pltpu.SemaphoreType.DMA,
        pltpu.SemaphoreType.DMA,
        pltpu.SemaphoreType.DMA,
        pltpu.SemaphoreType.DMA,
        pltpu.SemaphoreType.DMA,
    ],
)


def kernel(input, index):
    idx2 = jnp.transpose(index.astype(jnp.int32).reshape(NBLK, BLK, NC),
                         (0, 2, 1))
    perm = (8 * (jnp.arange(BLK) % (BLK // 8)) + jnp.arange(BLK) // (BLK // 8)
            ).astype(jnp.int32)
    idxp = jnp.take(idx2, perm, axis=2)
    zeros = jnp.zeros((S, DH), jnp.float32)
    ones = jnp.ones((BLK, L), jnp.float32)
    out = _sc_call(input, idx2, idxp, zeros, ones)
    return out.transpose(0, 2, 1, 3).reshape(N, NC * D)
